# baseline (speedup 1.0000x reference)
import functools

import jax
import jax.numpy as jnp
from jax import lax
from jax.experimental import pallas as pl
from jax.experimental.pallas import tpu as pltpu
from jax.experimental.pallas import tpu_sc as plsc

NSUB = 16
CHUNK = 64
NBUF = 3
CHUNK2 = 120
LANES = 16
DROW = 80


def _edge_pipeline(nch, start_idx, wait_idx, start_gather, wait_gather,
                   do_scatter, after_scatter=None):
  for b in range(NBUF):
    start_idx(b, b)
  wait_idx(0)
  start_gather(0, 0)
  plsc.subcore_barrier()

  def group(g, carry):
    for b in range(NBUF):
      i = g * NBUF + b
      nb = (b + 1) % NBUF

      @pl.when(i + 1 < nch)
      def _():
        wait_idx(nb)
        start_gather(nb, nb)

      wait_gather(b)
      do_scatter(b, b)
      if after_scatter is not None:
        after_scatter(b)

      @pl.when(i + NBUF < nch)
      def _():
        start_idx(i + NBUF, b)
    return carry

  lax.fori_loop(0, nch // NBUF, group, 0)
  plsc.subcore_barrier()


def _zero_vmem(ref, total):
  z = jnp.zeros((LANES,), ref.dtype)

  def st(t, carry):
    r = t // (ref.shape[-1] // LANES)
    cg = t % (ref.shape[-1] // LANES)
    ref[r, pl.ds(cg * LANES, LANES)] = z
    return carry

  lax.fori_loop(0, total // LANES, st, 0)


def _sc_pass_shared(n_pad, n_half, e_pad, dtype):
  ept = e_pad // NSUB
  nch = ept // CHUNK
  rps = n_pad // NSUB
  dcols = 8 * LANES

  def body(tab_lo, tab_hi, idx3, zin, zdeg,
           o_e_lo, o_e_hi, o_h_lo, o_h_hi, o_degs,
           idxr, rows, hist, idx80, acc1, acc2, degs, *sems):
    semi = sems[:NBUF]
    semg = sems[NBUF:2 * NBUF]
    semh = sems[2 * NBUF]
    c = lax.axis_index("c")
    s = lax.axis_index("s")
    srow = pl.multiple_of(s * rps, 8)
    pltpu.sync_copy(zin.at[pl.ds(srow, rps)], acc1.at[pl.ds(srow, rps)])
    pltpu.sync_copy(zin.at[pl.ds(srow, rps)], acc2.at[pl.ds(srow, rps)])

    for r in range(3):
      _zero_vmem(hist.at[r], DROW * dcols)
    iota = lax.iota(jnp.int32, LANES)
    for r in range(3):
      def sti(t, carry, r=r):
        idx80[r, pl.ds(t * LANES, LANES)] = iota + (r * DROW + t * LANES)
        return carry

      lax.fori_loop(0, DROW // LANES, sti, 0)

    @pl.when(s == 0)
    def _():
      pltpu.sync_copy(zdeg, degs)

    def start_idx(i, q):
      pltpu.async_copy(idx3.at[s * nch + i], idxr.at[q], semi[q])

    def wait_idx(q):
      pltpu.make_async_copy(idx3.at[0], idxr.at[q], semi[q]).wait()

    def start_gather(b, q):
      @pl.when(c == 0)
      def _():
        pltpu.async_copy(tab_lo.at[idxr.at[q, 0]], rows.at[b], semg[b])

      @pl.when(c == 1)
      def _():
        pltpu.async_copy(tab_hi.at[idxr.at[q, 0]], rows.at[b], semg[b])

    def wait_gather(b):
      pltpu.make_async_copy(tab_lo.at[idxr.at[0, 0]], rows.at[b],
                            semg[b]).wait()

    def do_scatter(b, q):
      pltpu.sync_copy(rows.at[b], acc1.at[idxr.at[q, 1]], add=True)
      pltpu.sync_copy(rows.at[b], acc2.at[idxr.at[q, 2]], add=True)

    ones = jnp.ones((LANES,), dtype)

    def hist_lists(q, lists):
      for r in lists:
        for t in range(CHUNK // LANES):
          v = idxr[q, r, pl.ds(t * LANES, LANES)]
          plsc.addupdate_scatter(hist.at[r], [v >> 7, v & 127], ones)

    def after_scatter(q):
      @pl.when(c == 0)
      def _():
        hist_lists(q, (0, 1))

      @pl.when(c == 1)
      def _():
        hist_lists(q, (2,))

    _edge_pipeline(nch, start_idx, wait_idx, start_gather, wait_gather,
                   do_scatter, after_scatter)

    @pl.when(c == 0)
    def _():
      for r in (0, 1):
        pltpu.async_copy(hist.at[r], degs.at[idx80.at[r]], semh,
                         add=True).wait()
      plsc.subcore_barrier()
      pltpu.sync_copy(acc1.at[pl.ds(srow, rps)], o_e_lo.at[pl.ds(srow, rps)])
      pltpu.sync_copy(acc2.at[pl.ds(srow, rps)], o_h_lo.at[pl.ds(srow, rps)])

      @pl.when(s == 0)
      def _():
        pltpu.sync_copy(degs.at[pl.ds(0, 2 * DROW)],
                        o_degs.at[pl.ds(0, 2 * DROW)])

    @pl.when(c == 1)
    def _():
      pltpu.async_copy(hist.at[2], degs.at[idx80.at[2]], semh,
                       add=True).wait()
      plsc.subcore_barrier()
      pltpu.sync_copy(acc1.at[pl.ds(srow, rps)], o_e_hi.at[pl.ds(srow, rps)])
      pltpu.sync_copy(acc2.at[pl.ds(srow, rps)], o_h_hi.at[pl.ds(srow, rps)])

      @pl.when(s == 0)
      def _():
        pltpu.sync_copy(degs.at[pl.ds(2 * DROW, DROW)],
                        o_degs.at[pl.ds(2 * DROW, DROW)])

  half = jax.ShapeDtypeStruct((n_pad, n_half), dtype)
  return pl.kernel(
      body,
      out_type=[half, half, half, half,
                jax.ShapeDtypeStruct((3 * DROW, 8 * LANES), dtype)],
      mesh=plsc.VectorSubcoreMesh(core_axis_name="c", subcore_axis_name="s"),
      scratch_types=[
          pltpu.VMEM((NBUF, 3, CHUNK), jnp.int32),
          pltpu.VMEM((NBUF, CHUNK, n_half), dtype),
          pltpu.VMEM((3, DROW, 8 * LANES), dtype),
          pltpu.VMEM((3, DROW), jnp.int32),
          pltpu.VMEM_SHARED((n_pad, n_half), dtype),
          pltpu.VMEM_SHARED((n_pad, n_half), dtype),
          pltpu.VMEM_SHARED((3 * DROW, 8 * LANES), dtype),
      ] + [pltpu.SemaphoreType.DMA] * (2 * NBUF + 1),
      compiler_params=pltpu.CompilerParams(use_tc_tiling_on_sc=False,
                                           needs_layout_passes=False),
  )


def _sc_pass(n_pad, n_cols, e_pad, dtype):
  ept = e_pad // NSUB
  nch = ept // CHUNK2
  rps = n_pad // NSUB

  def body(tab_a, tab_b, idx3, zinit,
           out_a, out_b, idxr, rows, acc, *sems):
    semi = sems[:NBUF]
    semg = sems[NBUF:]
    c = lax.axis_index("c")
    s = lax.axis_index("s")
    srow = pl.multiple_of(s * rps, 8)
    pltpu.sync_copy(zinit.at[pl.ds(srow, rps)],
                    acc.at[pl.ds(srow, rps)])

    def start_idx(i, q):
      pltpu.async_copy(idx3.at[s * nch + i], idxr.at[q], semi[q])

    def wait_idx(q):
      pltpu.make_async_copy(idx3.at[0], idxr.at[q], semi[q]).wait()

    def start_gather(b, q):
      @pl.when(c == 0)
      def _():
        pltpu.async_copy(tab_a.at[idxr.at[q, 1]], rows.at[b], semg[b])

      @pl.when(c == 1)
      def _():
        pltpu.async_copy(tab_b.at[idxr.at[q, 0]], rows.at[b], semg[b])

    def wait_gather(b):
      pltpu.make_async_copy(tab_a.at[idxr.at[0, 0]], rows.at[b],
                            semg[b]).wait()

    def do_scatter(b, q):
      @pl.when(c == 0)
      def _():
        pltpu.sync_copy(rows.at[b], acc.at[idxr.at[q, 0]], add=True)

      @pl.when(c == 1)
      def _():
        pltpu.sync_copy(rows.at[b], acc.at[idxr.at[q, 2]], add=True)

    _edge_pipeline(nch, start_idx, wait_idx, start_gather, wait_gather,
                   do_scatter)

    @pl.when(c == 0)
    def _():
      pltpu.sync_copy(acc.at[pl.ds(srow, rps)],
                      out_a.at[pl.ds(srow, rps)])

    @pl.when(c == 1)
    def _():
      pltpu.sync_copy(acc.at[pl.ds(srow, rps)],
                      out_b.at[pl.ds(srow, rps)])

  shape = jax.ShapeDtypeStruct((n_pad, n_cols), dtype)
  return pl.kernel(
      body,
      out_type=[shape, shape],
      mesh=plsc.VectorSubcoreMesh(core_axis_name="c", subcore_axis_name="s"),
      scratch_types=[
          pltpu.VMEM((NBUF, 3, CHUNK2), jnp.int32),
          pltpu.VMEM((NBUF, CHUNK2, n_cols), dtype),
          pltpu.VMEM_SHARED((n_pad, n_cols), dtype),
      ] + [pltpu.SemaphoreType.DMA] * (2 * NBUF),
      compiler_params=pltpu.CompilerParams(use_tc_tiling_on_sc=False,
                                           needs_layout_passes=False),
  )


def _relu_scale(acc, deg):
  return jnp.maximum(acc / jnp.maximum(deg, 1.0), 0.0)


def _l2n(v):
  n = jnp.sqrt(jnp.sum(v * v, axis=1, keepdims=True))
  return v / jnp.maximum(n, 1e-12)


def _tc1_body(d, e_lo_ref, e_hi_ref, h_lo_ref, h_hi_ref, dege_ref, deg_ref,
              xe_ref, g1_ref):
  hd = d // 2

  def join(lo, hi, deg):
    return _relu_scale(jnp.concatenate([lo[:, :hd], hi[:, :hd]], axis=1), deg)

  xe_ref[...] = join(e_lo_ref[...], e_hi_ref[...], dege_ref[...])
  g1_ref[...] = _l2n(join(h_lo_ref[...], h_hi_ref[...], deg_ref[...]))


def _tc2_body(acc_v_ref, acc_h2_ref, x_ref, g1_ref, degv_ref, deg_ref,
              out_ref):
  x_v = _l2n(_relu_scale(acc_v_ref[...], degv_ref[...]))
  h2 = _l2n(_relu_scale(acc_h2_ref[...], deg_ref[...]))
  cat = jnp.concatenate([x_ref[...], x_v, g1_ref[...], h2], axis=1)
  out_ref[...] = _l2n(cat)


def kernel(x, edge):
  n, d = x.shape
  e = edge.shape[1]
  dtype = x.dtype
  hd = d // 2
  n_pad = ((n + 1 + NSUB * 8 - 1) // (NSUB * 8)) * (NSUB * 8)
  import math
  egrp = NSUB * NBUF * math.lcm(CHUNK, CHUNK2)
  e_pad = ((e + egrp - 1) // egrp) * egrp
  assert DROW * 8 * LANES >= n_pad

  x_lo = jnp.zeros((n_pad, hd), dtype).at[:n].set(x[:, :hd])
  x_hi = jnp.zeros((n_pad, hd), dtype).at[:n].set(x[:, hd:])

  pad = jnp.full((e_pad - e,), n, jnp.int32)
  src = jnp.concatenate([edge[0], pad])
  eid = jnp.concatenate([edge[1], pad])
  dst = jnp.concatenate([edge[2], pad])
  zinit = jnp.zeros((n_pad, d), dtype)

  def triplet(ch):
    return jnp.stack([a.reshape(-1, ch) for a in (src, eid, dst)], axis=1)

  trip1 = triplet(CHUNK)
  trip2 = triplet(CHUNK2)

  zdeg = jnp.zeros((3 * DROW, 8 * LANES), dtype)
  sc1 = _sc_pass_shared(n_pad, hd, e_pad, dtype)
  e_lo, e_hi, h_lo, h_hi, degs = sc1(x_lo, x_hi, trip1,
                                     zinit[:, :hd], zdeg)
  degs = degs.reshape(3, DROW * 8 * LANES)[:, :n_pad, None]
  deg_v, deg_e, deg = degs[0], degs[1], degs[2]

  br = max(b for b in range(8, n_pad + 1, 8)
           if n_pad % b == 0 and b * d * 4 <= 1536 * 1024)
  grid = (n_pad // br,)
  spec = pl.BlockSpec((br, d), lambda i: (i, 0))
  hspec = pl.BlockSpec((br, hd), lambda i: (i, 0))
  dspec = pl.BlockSpec((br, 1), lambda i: (i, 0))
  xe, g1 = pl.pallas_call(
      functools.partial(_tc1_body, d),
      grid=grid,
      in_specs=[hspec] * 4 + [dspec] * 2,
      out_specs=[spec, spec],
      out_shape=[jax.ShapeDtypeStruct((n_pad, d), dtype)] * 2,
  )(e_lo, e_hi, h_lo, h_hi, deg_e, deg)

  sc2 = _sc_pass(n_pad, d, e_pad, dtype)
  acc_v, acc_h2 = sc2(xe, g1, trip2, zinit)

  x2 = jnp.zeros((n_pad, d), dtype).at[:n].set(x)
  out = pl.pallas_call(
      _tc2_body,
      grid=grid,
      in_specs=[spec, spec, spec, spec, dspec, dspec],
      out_specs=pl.BlockSpec((br, 4 * d), lambda i: (i, 0)),
      out_shape=jax.ShapeDtypeStruct((n_pad, 4 * d), dtype),
  )(acc_v, acc_h2, x2, g1, deg_v, deg)
  return out[:n]


# speedup vs baseline: 1.0477x; 1.0477x over previous
import functools

import jax
import jax.numpy as jnp
from jax import lax
from jax.experimental import pallas as pl
from jax.experimental.pallas import tpu as pltpu
from jax.experimental.pallas import tpu_sc as plsc

NSUB = 16
CHUNK = 80
NBUF = 3
CHUNK2 = 120
LANES = 16
DROW = 80


def _edge_pipeline(nch, start_idx, wait_idx, start_gather, wait_gather,
                   do_scatter, after_scatter=None):
  for b in range(NBUF):
    start_idx(b, b)
  wait_idx(0)
  start_gather(0, 0)
  plsc.subcore_barrier()

  def group(g, carry):
    for b in range(NBUF):
      i = g * NBUF + b
      nb = (b + 1) % NBUF

      @pl.when(i + 1 < nch)
      def _():
        wait_idx(nb)
        start_gather(nb, nb)

      wait_gather(b)
      do_scatter(b, b)
      if after_scatter is not None:
        after_scatter(b)

      @pl.when(i + NBUF < nch)
      def _():
        start_idx(i + NBUF, b)
    return carry

  lax.fori_loop(0, nch // NBUF, group, 0)
  plsc.subcore_barrier()


def _zero_vmem(ref, total):
  z = jnp.zeros((LANES,), ref.dtype)

  def st(t, carry):
    r = t // (ref.shape[-1] // LANES)
    cg = t % (ref.shape[-1] // LANES)
    ref[r, pl.ds(cg * LANES, LANES)] = z
    return carry

  lax.fori_loop(0, total // LANES, st, 0)


def _sc_pass_shared(n_pad, n_half, e_pad, dtype):
  ept = e_pad // NSUB
  nch = ept // CHUNK
  rps = n_pad // NSUB
  dcols = 8 * LANES

  def body(tab_lo, tab_hi, idx3, zin, zdeg,
           o_e_lo, o_e_hi, o_h_lo, o_h_hi, o_degs,
           idxr, rows, hist, idx80, acc1, acc2, degs, *sems):
    semi = sems[:NBUF]
    semg = sems[NBUF:2 * NBUF]
    semh = sems[2 * NBUF]
    c = lax.axis_index("c")
    s = lax.axis_index("s")
    srow = pl.multiple_of(s * rps, 8)
    pltpu.sync_copy(zin.at[pl.ds(srow, rps)], acc1.at[pl.ds(srow, rps)])
    pltpu.sync_copy(zin.at[pl.ds(srow, rps)], acc2.at[pl.ds(srow, rps)])

    @pl.when(c == 0)
    def _():
      for r in range(3):
        _zero_vmem(hist.at[r], DROW * dcols)
      iota = lax.iota(jnp.int32, LANES)
      for r in range(3):
        def sti(t, carry, r=r):
          idx80[r, pl.ds(t * LANES, LANES)] = iota + (r * DROW + t * LANES)
          return carry

        lax.fori_loop(0, DROW // LANES, sti, 0)

      @pl.when(s == 0)
      def _():
        pltpu.sync_copy(zdeg, degs)

    def start_idx(i, q):
      pltpu.async_copy(idx3.at[s * nch + i], idxr.at[q], semi[q])

    def wait_idx(q):
      pltpu.make_async_copy(idx3.at[0], idxr.at[q], semi[q]).wait()

    def start_gather(b, q):
      @pl.when(c == 0)
      def _():
        pltpu.async_copy(tab_lo.at[idxr.at[q, 0]], rows.at[b], semg[b])

      @pl.when(c == 1)
      def _():
        pltpu.async_copy(tab_hi.at[idxr.at[q, 0]], rows.at[b], semg[b])

    def wait_gather(b):
      pltpu.make_async_copy(tab_lo.at[idxr.at[0, 0]], rows.at[b],
                            semg[b]).wait()

    def do_scatter(b, q):
      pltpu.sync_copy(rows.at[b], acc1.at[idxr.at[q, 1]], add=True)
      pltpu.sync_copy(rows.at[b], acc2.at[idxr.at[q, 2]], add=True)

    ones = jnp.ones((LANES,), dtype)

    def after_scatter(q):
      @pl.when(c == 0)
      def _():
        for r in range(3):
          for t in range(CHUNK // LANES):
            v = idxr[q, r, pl.ds(t * LANES, LANES)]
            plsc.addupdate_scatter(hist.at[r], [v >> 7, v & 127], ones)

    _edge_pipeline(nch, start_idx, wait_idx, start_gather, wait_gather,
                   do_scatter, after_scatter)

    @pl.when(c == 0)
    def _():
      for r in range(3):
        pltpu.async_copy(hist.at[r], degs.at[idx80.at[r]], semh,
                         add=True).wait()
      plsc.subcore_barrier()
      pltpu.sync_copy(acc1.at[pl.ds(srow, rps)], o_e_lo.at[pl.ds(srow, rps)])
      pltpu.sync_copy(acc2.at[pl.ds(srow, rps)], o_h_lo.at[pl.ds(srow, rps)])

      @pl.when(s == 0)
      def _():
        pltpu.sync_copy(degs, o_degs)

    @pl.when(c == 1)
    def _():
      plsc.subcore_barrier()
      pltpu.sync_copy(acc1.at[pl.ds(srow, rps)], o_e_hi.at[pl.ds(srow, rps)])
      pltpu.sync_copy(acc2.at[pl.ds(srow, rps)], o_h_hi.at[pl.ds(srow, rps)])

  half = jax.ShapeDtypeStruct((n_pad, n_half), dtype)
  return pl.kernel(
      body,
      out_type=[half, half, half, half,
                jax.ShapeDtypeStruct((3 * DROW, 8 * LANES), dtype)],
      mesh=plsc.VectorSubcoreMesh(core_axis_name="c", subcore_axis_name="s"),
      scratch_types=[
          pltpu.VMEM((NBUF, 3, CHUNK), jnp.int32),
          pltpu.VMEM((NBUF, CHUNK, n_half), dtype),
          pltpu.VMEM((3, DROW, 8 * LANES), dtype),
          pltpu.VMEM((3, DROW), jnp.int32),
          pltpu.VMEM_SHARED((n_pad, n_half), dtype),
          pltpu.VMEM_SHARED((n_pad, n_half), dtype),
          pltpu.VMEM_SHARED((3 * DROW, 8 * LANES), dtype),
      ] + [pltpu.SemaphoreType.DMA] * (2 * NBUF + 1),
      compiler_params=pltpu.CompilerParams(use_tc_tiling_on_sc=False,
                                           needs_layout_passes=False),
  )


def _sc_pass(n_pad, n_cols, e_pad, dtype):
  ept = e_pad // NSUB
  nch = ept // CHUNK2
  rps = n_pad // NSUB

  def body(tab_a, tab_b, idx3, zinit,
           out_a, out_b, idxr, rows, acc, *sems):
    semi = sems[:NBUF]
    semg = sems[NBUF:]
    c = lax.axis_index("c")
    s = lax.axis_index("s")
    srow = pl.multiple_of(s * rps, 8)
    pltpu.sync_copy(zinit.at[pl.ds(srow, rps)],
                    acc.at[pl.ds(srow, rps)])

    def start_idx(i, q):
      pltpu.async_copy(idx3.at[s * nch + i], idxr.at[q], semi[q])

    def wait_idx(q):
      pltpu.make_async_copy(idx3.at[0], idxr.at[q], semi[q]).wait()

    def start_gather(b, q):
      @pl.when(c == 0)
      def _():
        pltpu.async_copy(tab_a.at[idxr.at[q, 1]], rows.at[b], semg[b])

      @pl.when(c == 1)
      def _():
        pltpu.async_copy(tab_b.at[idxr.at[q, 0]], rows.at[b], semg[b])

    def wait_gather(b):
      pltpu.make_async_copy(tab_a.at[idxr.at[0, 0]], rows.at[b],
                            semg[b]).wait()

    def do_scatter(b, q):
      @pl.when(c == 0)
      def _():
        pltpu.sync_copy(rows.at[b], acc.at[idxr.at[q, 0]], add=True)

      @pl.when(c == 1)
      def _():
        pltpu.sync_copy(rows.at[b], acc.at[idxr.at[q, 2]], add=True)

    _edge_pipeline(nch, start_idx, wait_idx, start_gather, wait_gather,
                   do_scatter)

    @pl.when(c == 0)
    def _():
      pltpu.sync_copy(acc.at[pl.ds(srow, rps)],
                      out_a.at[pl.ds(srow, rps)])

    @pl.when(c == 1)
    def _():
      pltpu.sync_copy(acc.at[pl.ds(srow, rps)],
                      out_b.at[pl.ds(srow, rps)])

  shape = jax.ShapeDtypeStruct((n_pad, n_cols), dtype)
  return pl.kernel(
      body,
      out_type=[shape, shape],
      mesh=plsc.VectorSubcoreMesh(core_axis_name="c", subcore_axis_name="s"),
      scratch_types=[
          pltpu.VMEM((NBUF, 3, CHUNK2), jnp.int32),
          pltpu.VMEM((NBUF, CHUNK2, n_cols), dtype),
          pltpu.VMEM_SHARED((n_pad, n_cols), dtype),
      ] + [pltpu.SemaphoreType.DMA] * (2 * NBUF),
      compiler_params=pltpu.CompilerParams(use_tc_tiling_on_sc=False,
                                           needs_layout_passes=False),
  )


def _relu_scale(acc, deg):
  return jnp.maximum(acc / jnp.maximum(deg, 1.0), 0.0)


def _l2n(v):
  n = jnp.sqrt(jnp.sum(v * v, axis=1, keepdims=True))
  return v / jnp.maximum(n, 1e-12)


def _tc1_body(d, e_lo_ref, e_hi_ref, h_lo_ref, h_hi_ref, dege_ref, deg_ref,
              xe_ref, g1_ref):
  hd = d // 2

  def join(lo, hi, deg):
    return _relu_scale(jnp.concatenate([lo[:, :hd], hi[:, :hd]], axis=1), deg)

  xe_ref[...] = join(e_lo_ref[...], e_hi_ref[...], dege_ref[...])
  g1_ref[...] = _l2n(join(h_lo_ref[...], h_hi_ref[...], deg_ref[...]))


def _tc2_body(acc_v_ref, acc_h2_ref, x_ref, g1_ref, degv_ref, deg_ref,
              out_ref):
  x_v = _l2n(_relu_scale(acc_v_ref[...], degv_ref[...]))
  h2 = _l2n(_relu_scale(acc_h2_ref[...], deg_ref[...]))
  cat = jnp.concatenate([x_ref[...], x_v, g1_ref[...], h2], axis=1)
  out_ref[...] = _l2n(cat)


def kernel(x, edge):
  n, d = x.shape
  e = edge.shape[1]
  dtype = x.dtype
  hd = d // 2
  n_pad = ((n + 1 + NSUB * 8 - 1) // (NSUB * 8)) * (NSUB * 8)
  import math
  egrp = NSUB * NBUF * math.lcm(CHUNK, CHUNK2)
  e_pad = ((e + egrp - 1) // egrp) * egrp
  assert DROW * 8 * LANES >= n_pad

  x_lo = jnp.zeros((n_pad, hd), dtype).at[:n].set(x[:, :hd])
  x_hi = jnp.zeros((n_pad, hd), dtype).at[:n].set(x[:, hd:])

  pad = jnp.full((e_pad - e,), n, jnp.int32)
  src = jnp.concatenate([edge[0], pad])
  eid = jnp.concatenate([edge[1], pad])
  dst = jnp.concatenate([edge[2], pad])
  zinit = jnp.zeros((n_pad, d), dtype)

  def triplet(ch):
    return jnp.stack([a.reshape(-1, ch) for a in (src, eid, dst)], axis=1)

  trip1 = triplet(CHUNK)
  trip2 = triplet(CHUNK2)

  zdeg = jnp.zeros((3 * DROW, 8 * LANES), dtype)
  sc1 = _sc_pass_shared(n_pad, hd, e_pad, dtype)
  e_lo, e_hi, h_lo, h_hi, degs = sc1(x_lo, x_hi, trip1,
                                     zinit[:, :hd], zdeg)
  degs = degs.reshape(3, DROW * 8 * LANES)[:, :n_pad, None]
  deg_v, deg_e, deg = degs[0], degs[1], degs[2]

  br = max(b for b in range(8, n_pad + 1, 8)
           if n_pad % b == 0 and b * d * 4 <= 1536 * 1024)
  grid = (n_pad // br,)
  spec = pl.BlockSpec((br, d), lambda i: (i, 0))
  hspec = pl.BlockSpec((br, hd), lambda i: (i, 0))
  dspec = pl.BlockSpec((br, 1), lambda i: (i, 0))
  xe, g1 = pl.pallas_call(
      functools.partial(_tc1_body, d),
      grid=grid,
      in_specs=[hspec] * 4 + [dspec] * 2,
      out_specs=[spec, spec],
      out_shape=[jax.ShapeDtypeStruct((n_pad, d), dtype)] * 2,
  )(e_lo, e_hi, h_lo, h_hi, deg_e, deg)

  sc2 = _sc_pass(n_pad, d, e_pad, dtype)
  acc_v, acc_h2 = sc2(xe, g1, trip2, zinit)

  x2 = jnp.zeros((n_pad, d), dtype).at[:n].set(x)
  out = pl.pallas_call(
      _tc2_body,
      grid=grid,
      in_specs=[spec, spec, spec, spec, dspec, dspec],
      out_specs=pl.BlockSpec((br, 4 * d), lambda i: (i, 0)),
      out_shape=jax.ShapeDtypeStruct((n_pad, 4 * d), dtype),
  )(acc_v, acc_h2, x2, g1, deg_v, deg)
  return out[:n]
